# baseline (speedup 1.0000x reference)
import functools

import jax
import jax.numpy as jnp
import numpy as np
from jax import lax
from jax.experimental import pallas as pl
from jax.experimental.pallas import tpu as pltpu
from jax.experimental.pallas import tpu_sc as plsc

_B, _L, _D = 1024, 200, 32
_SAMPLE_NUM = 10
_LM1 = _L - 1
_NNEG = _LM1 * _SAMPLE_NUM

_NC = 2
_NS = 16
_NW = _NC * _NS


def _sample_idx_table(L, sample_num, seed=0):
    rng = np.random.RandomState(seed)
    all_idx = [
        rng.choice([j for j in range(L) if j != idx_], size=sample_num, replace=False)
        for idx_ in range(L)
    ]
    return np.stack(all_idx[1:], axis=0).astype(np.int32)


def _routing_tables():
    flat = _sample_idx_table(_L, _SAMPLE_NUM).reshape(-1)
    counts = np.bincount(flat, minlength=_L).astype(np.int32)
    dests = [[] for _ in range(_L)]
    for k, j in enumerate(flat):
        dests[j].append(k)
    for j in range(1, _L):
        dests[j].append(_NNEG + j - 1)

    weight = counts + (np.arange(_L) >= 1)
    order = np.argsort(-weight, kind="stable")
    loads = np.zeros(_NW, np.int64)
    assign = [[] for _ in range(_NW)]
    for j in order:
        w = int(np.argmin(loads))
        loads[w] += int(weight[j])
        assign[w].append(int(j))

    pairs = [[] for _ in range(_NW)]
    for w, slabs in enumerate(assign):
        for j in slabs:
            for d in dests[j]:
                pairs[w].append((j, d))
    maxp = max(len(p) for p in pairs)
    width = -(-maxp // 16) * 16
    src_t = np.zeros((_NW, width), np.int32)
    dst_t = np.zeros((_NW, width), np.int32)
    npair = np.zeros((_NW, 16), np.int32)
    for w, p in enumerate(pairs):
        npair[w, 0] = len(p)
        for i, (j, d) in enumerate(p):
            src_t[w, i] = j
            dst_t[w, i] = d
    return src_t, dst_t, npair, width


_SRC_T, _DST_T, _NPAIR, _WIDTH = _routing_tables()

_mesh = plsc.VectorSubcoreMesh(core_axis_name="c", subcore_axis_name="s")


@functools.partial(
    pl.kernel,
    mesh=_mesh,
    out_type=(
        jax.ShapeDtypeStruct((_LM1, _D, _B), jnp.float32),
        jax.ShapeDtypeStruct((_NNEG, _D, _B), jnp.float32),
    ),
    scratch_types=[
        pltpu.VMEM((_NW, _WIDTH), jnp.int32),
        pltpu.VMEM((_NW, _WIDTH), jnp.int32),
        pltpu.VMEM((_NW, 16), jnp.int32),
        pltpu.SemaphoreType.DMA,
    ],
    compiler_params=pltpu.CompilerParams(needs_layout_passes=False),
)
def _sc_route(
    x_hbm, src_hbm, dst_hbm, np_hbm, pos_hbm, neg_hbm, src_v, dst_v, np_v, wsem
):
    w = lax.axis_index("s") * _NC + lax.axis_index("c")
    pltpu.sync_copy(src_hbm, src_v)
    pltpu.sync_copy(dst_hbm, dst_v)
    pltpu.sync_copy(np_hbm, np_v)
    lanes = lax.broadcasted_iota(jnp.int32, (16,), 0)

    def lane_of(vec, lane):
        return jnp.sum(jnp.where(lanes == lane, vec, 0))

    n = lane_of(np_v[w, pl.ds(0, 16)], 0)

    def pair_body(i, carry):
        base = (i // 16) * 16
        lane = i - base
        j = lane_of(src_v[w, pl.ds(base, 16)], lane)
        d = lane_of(dst_v[w, pl.ds(base, 16)], lane)

        @pl.when(d < _NNEG)
        def _():
            pltpu.async_copy(x_hbm.at[j], neg_hbm.at[d], wsem)

        @pl.when(d >= _NNEG)
        def _():
            pltpu.async_copy(x_hbm.at[j], pos_hbm.at[d - _NNEG], wsem)

        return carry

    lax.fori_loop(0, n, pair_body, 0)

    def drain(i, carry):
        pltpu.make_async_copy(x_hbm.at[0], neg_hbm.at[0], wsem).wait()
        return carry

    lax.fori_loop(0, n, drain, 0)


def kernel(inputs):
    b, l, d = inputs.shape
    x_t = jnp.transpose(inputs, (1, 2, 0))
    pos_t, neg_t = _sc_route(
        x_t, jnp.asarray(_SRC_T), jnp.asarray(_DST_T), jnp.asarray(_NPAIR)
    )
    pos = jnp.transpose(pos_t, (2, 0, 1))
    neg = jnp.transpose(
        neg_t.reshape(_LM1, _SAMPLE_NUM, d, b), (3, 0, 1, 2)
    )
    return pos, neg


# speedup vs baseline: 49.2169x; 49.2169x over previous
import functools

import jax
import jax.numpy as jnp
import numpy as np
from jax import lax
from jax.experimental import pallas as pl
from jax.experimental.pallas import tpu as pltpu
from jax.experimental.pallas import tpu_sc as plsc

_B, _L, _D = 1024, 200, 32
_SAMPLE_NUM = 10
_LM1 = _L - 1
_NNEG = _LM1 * _SAMPLE_NUM

_NC = 2
_NS = 16
_NW = _NC * _NS


def _sample_idx_table(L, sample_num, seed=0):
    rng = np.random.RandomState(seed)
    all_idx = [
        rng.choice([j for j in range(L) if j != idx_], size=sample_num, replace=False)
        for idx_ in range(L)
    ]
    return np.stack(all_idx[1:], axis=0).astype(np.int32)


def _routing_tables():
    flat = _sample_idx_table(_L, _SAMPLE_NUM).reshape(-1)
    counts = np.bincount(flat, minlength=_L).astype(np.int32)
    assert int(counts.max()) <= 31
    dtbl = np.zeros((_L, 32), np.int32)
    dtbl[:, 0] = counts
    fill = np.ones((_L,), np.int32)
    for k, j in enumerate(flat):
        dtbl[j, fill[j]] = k
        fill[j] += 1

    weight = counts
    order = np.argsort(-weight, kind="stable")
    loads = np.zeros(_NW, np.int64)
    assign = [[] for _ in range(_NW)]
    for j in order:
        w = int(np.argmin(loads))
        loads[w] += int(weight[j])
        assign[w].append(int(j))
    max_slabs = max(len(a) for a in assign)
    assert max_slabs <= 15
    wtbl = np.zeros((_NW, 16), np.int32)
    for w, a in enumerate(assign):
        wtbl[w, 0] = len(a)
        wtbl[w, 1 : 1 + len(a)] = a
    return dtbl, wtbl


_DTBL, _WTBL = _routing_tables()

_mesh = plsc.VectorSubcoreMesh(core_axis_name="c", subcore_axis_name="s")


@functools.partial(
    pl.kernel,
    mesh=_mesh,
    out_type=jax.ShapeDtypeStruct((_NNEG, _D, _B), jnp.float32),
    scratch_types=[
        pltpu.VMEM((_L, 32), jnp.int32),
        pltpu.VMEM((_NW, 16), jnp.int32),
        pltpu.VMEM((2, _D, _B), jnp.float32),
        pltpu.SemaphoreType.DMA,
        pltpu.SemaphoreType.DMA,
        pltpu.SemaphoreType.DMA,
        pltpu.SemaphoreType.DMA,
    ],
    compiler_params=pltpu.CompilerParams(needs_layout_passes=False),
)
def _sc_route(
    x_hbm, dtbl_hbm, wtbl_hbm, neg_hbm,
    dtbl_v, wtbl_v, slab_v, lsem0, lsem1, wsem0, wsem1,
):
    w = lax.axis_index("s") * _NC + lax.axis_index("c")
    pltpu.sync_copy(dtbl_hbm, dtbl_v)
    pltpu.sync_copy(wtbl_hbm, wtbl_v)
    lanes = lax.broadcasted_iota(jnp.int32, (16,), 0)
    wv = wtbl_v[w, pl.ds(0, 16)]

    def lane_of(vec, lane):
        return jnp.sum(jnp.where(lanes == lane, vec, 0))

    nslab = lane_of(wv, 0)
    j0 = lane_of(wv, 1)
    pltpu.async_copy(x_hbm.at[j0], slab_v.at[0], lsem0)

    def drain(sem, n):
        def one(_, c):
            pltpu.make_async_copy(slab_v.at[0], neg_hbm.at[0], sem).wait()
            return c

        lax.fori_loop(0, n, one, 0)

    def slab_body(si, prev_writes):
        p = si % 2
        j = lane_of(wv, si + 1)
        rv1 = dtbl_v[j, pl.ds(0, 16)]
        rv2 = dtbl_v[j, pl.ds(16, 16)]
        cnt = lane_of(rv1, 0)

        @pl.when(p == 0)
        def _():
            pltpu.make_async_copy(x_hbm.at[j], slab_v.at[0], lsem0).wait()

        @pl.when(p == 1)
        def _():
            pltpu.make_async_copy(x_hbm.at[j], slab_v.at[1], lsem1).wait()

        def dest_body(c, carry2):
            cc = c + 1
            rv = jnp.where(cc < 16, rv1, rv2)
            lane = jnp.where(cc < 16, cc, cc - 16)
            d = jnp.sum(jnp.where(lanes == lane, rv, 0))

            @pl.when(p == 0)
            def _():
                pltpu.async_copy(slab_v.at[0], neg_hbm.at[d], wsem0)

            @pl.when(p == 1)
            def _():
                pltpu.async_copy(slab_v.at[1], neg_hbm.at[d], wsem1)

            return carry2

        lax.fori_loop(0, cnt, dest_body, 0)
        writes = cnt

        @pl.when(p == 0)
        def _():
            drain(wsem1, prev_writes)

        @pl.when(p == 1)
        def _():
            drain(wsem0, prev_writes)

        @pl.when(si + 1 < nslab)
        def _():
            jn = lane_of(wv, si + 2)

            @pl.when(p == 0)
            def _():
                pltpu.async_copy(x_hbm.at[jn], slab_v.at[1], lsem1)

            @pl.when(p == 1)
            def _():
                pltpu.async_copy(x_hbm.at[jn], slab_v.at[0], lsem0)

        return writes

    last_writes = lax.fori_loop(0, nslab, slab_body, 0)

    @pl.when((nslab % 2) == 1)
    def _():
        drain(wsem0, last_writes)

    @pl.when((nslab % 2) == 0)
    def _():
        drain(wsem1, last_writes)


def _pos_body(x_ref, pos_ref):
    pos_ref[...] = x_ref[...]


_pos_copy = pl.pallas_call(
    _pos_body,
    grid=(_LM1,),
    in_specs=[pl.BlockSpec((1, _D, _B), lambda i: (i + 1, 0, 0))],
    out_specs=pl.BlockSpec((1, _D, _B), lambda i: (i, 0, 0)),
    out_shape=jax.ShapeDtypeStruct((_LM1, _D, _B), jnp.float32),
)


def kernel(inputs):
    b, l, d = inputs.shape
    x_t = jnp.transpose(inputs, (1, 2, 0))
    neg_t = _sc_route(x_t, jnp.asarray(_DTBL), jnp.asarray(_WTBL))
    pos_t = _pos_copy(x_t)
    pos = jnp.transpose(pos_t, (2, 0, 1))
    neg = jnp.transpose(
        neg_t.reshape(_LM1, _SAMPLE_NUM, d, b), (3, 0, 1, 2)
    )
    return pos, neg


# speedup vs baseline: 64.0445x; 1.3013x over previous
import functools

import jax
import jax.numpy as jnp
import numpy as np
from jax import lax
from jax.experimental import pallas as pl
from jax.experimental.pallas import tpu as pltpu
from jax.experimental.pallas import tpu_sc as plsc

_B, _L, _D = 1024, 200, 32
_SAMPLE_NUM = 10
_LM1 = _L - 1
_NNEG = _LM1 * _SAMPLE_NUM

_NC = 2
_NS = 16
_NW = _NC * _NS


def _sample_idx_table(L, sample_num, seed=0):
    rng = np.random.RandomState(seed)
    all_idx = [
        rng.choice([j for j in range(L) if j != idx_], size=sample_num, replace=False)
        for idx_ in range(L)
    ]
    return np.stack(all_idx[1:], axis=0).astype(np.int32)


def _routing_tables():
    flat = _sample_idx_table(_L, _SAMPLE_NUM).reshape(-1)
    counts = np.bincount(flat, minlength=_L).astype(np.int32)
    assert int(counts.max()) <= 31
    dtbl = np.zeros((_L, 32), np.int32)
    dtbl[:, 0] = counts
    fill = np.ones((_L,), np.int32)
    for k, j in enumerate(flat):
        dtbl[j, fill[j]] = k
        fill[j] += 1

    weight = counts
    order = np.argsort(-weight, kind="stable")
    loads = np.zeros(_NW, np.int64)
    assign = [[] for _ in range(_NW)]
    for j in order:
        w = int(np.argmin(loads))
        loads[w] += int(weight[j])
        assign[w].append(int(j))
    max_slabs = max(len(a) for a in assign)
    assert max_slabs <= 15
    wtbl = np.zeros((_NW, 16), np.int32)
    for w, a in enumerate(assign):
        wtbl[w, 0] = len(a)
        wtbl[w, 1 : 1 + len(a)] = a
    return dtbl, wtbl


_DTBL, _WTBL = _routing_tables()

_mesh = plsc.VectorSubcoreMesh(core_axis_name="c", subcore_axis_name="s")


@functools.partial(
    pl.kernel,
    mesh=_mesh,
    out_type=jax.ShapeDtypeStruct((_NNEG, _D, _B), jnp.float32),
    scratch_types=[
        pltpu.VMEM((_L, 32), jnp.int32),
        pltpu.VMEM((_NW, 16), jnp.int32),
        pltpu.VMEM((2, _D, _B), jnp.float32),
        pltpu.SemaphoreType.DMA,
        pltpu.SemaphoreType.DMA,
        pltpu.SemaphoreType.DMA,
        pltpu.SemaphoreType.DMA,
    ],
    compiler_params=pltpu.CompilerParams(needs_layout_passes=False),
)
def _sc_route(
    x_hbm, dtbl_hbm, wtbl_hbm, neg_hbm,
    dtbl_v, wtbl_v, slab_v, lsem0, lsem1, wsem0, wsem1,
):
    w = lax.axis_index("s") * _NC + lax.axis_index("c")
    pltpu.sync_copy(dtbl_hbm, dtbl_v)
    pltpu.sync_copy(wtbl_hbm, wtbl_v)
    lanes = lax.broadcasted_iota(jnp.int32, (16,), 0)
    wv = wtbl_v[w, pl.ds(0, 16)]

    def lane_of(vec, lane):
        return jnp.sum(jnp.where(lanes == lane, vec, 0))

    nslab = lane_of(wv, 0)
    j0 = lane_of(wv, 1)
    pltpu.async_copy(x_hbm.at[j0], slab_v.at[0], lsem0)

    def drain(sem, n):
        def one(_, c):
            pltpu.make_async_copy(slab_v.at[0], neg_hbm.at[0], sem).wait()
            return c

        lax.fori_loop(0, n, one, 0)

    def slab_body(si, prev_writes):
        p = si % 2
        j = lane_of(wv, si + 1)
        rv1 = dtbl_v[j, pl.ds(0, 16)]
        rv2 = dtbl_v[j, pl.ds(16, 16)]
        cnt = lane_of(rv1, 0)

        @pl.when(p == 0)
        def _():
            pltpu.make_async_copy(x_hbm.at[j], slab_v.at[0], lsem0).wait()

        @pl.when(p == 1)
        def _():
            pltpu.make_async_copy(x_hbm.at[j], slab_v.at[1], lsem1).wait()

        def dest_body(c, carry2):
            cc = c + 1
            rv = jnp.where(cc < 16, rv1, rv2)
            lane = jnp.where(cc < 16, cc, cc - 16)
            d = jnp.sum(jnp.where(lanes == lane, rv, 0))

            @pl.when(p == 0)
            def _():
                pltpu.async_copy(slab_v.at[0], neg_hbm.at[d], wsem0)

            @pl.when(p == 1)
            def _():
                pltpu.async_copy(slab_v.at[1], neg_hbm.at[d], wsem1)

            return carry2

        lax.fori_loop(0, cnt, dest_body, 0)
        writes = cnt

        @pl.when(p == 0)
        def _():
            drain(wsem1, prev_writes)

        @pl.when(p == 1)
        def _():
            drain(wsem0, prev_writes)

        @pl.when(si + 1 < nslab)
        def _():
            jn = lane_of(wv, si + 2)

            @pl.when(p == 0)
            def _():
                pltpu.async_copy(x_hbm.at[jn], slab_v.at[1], lsem1)

            @pl.when(p == 1)
            def _():
                pltpu.async_copy(x_hbm.at[jn], slab_v.at[0], lsem0)

        return writes

    last_writes = lax.fori_loop(0, nslab, slab_body, 0)

    @pl.when((nslab % 2) == 1)
    def _():
        drain(wsem0, last_writes)

    @pl.when((nslab % 2) == 0)
    def _():
        drain(wsem1, last_writes)


def _pos_body(x_ref, pos_ref):
    pos_ref[...] = x_ref[pl.ds(_D, _LM1 * _D), :]


_pos_copy = pl.pallas_call(
    _pos_body,
    grid=(8,),
    in_specs=[pl.BlockSpec((_L * _D, _B // 8), lambda i: (0, i))],
    out_specs=pl.BlockSpec((_LM1 * _D, _B // 8), lambda i: (0, i)),
    out_shape=jax.ShapeDtypeStruct((_LM1 * _D, _B), jnp.float32),
)


def kernel(inputs):
    b, l, d = inputs.shape
    x_t = jnp.transpose(inputs, (1, 2, 0))
    neg_t = _sc_route(x_t, jnp.asarray(_DTBL), jnp.asarray(_WTBL))
    pos_t = _pos_copy(x_t.reshape(l * d, b)).reshape(_LM1, d, b)
    pos = jnp.transpose(pos_t, (2, 0, 1))
    neg = jnp.transpose(
        neg_t.reshape(_LM1, _SAMPLE_NUM, d, b), (3, 0, 1, 2)
    )
    return pos, neg


# speedup vs baseline: 66.5640x; 1.0393x over previous
import functools

import jax
import jax.numpy as jnp
import numpy as np
from jax import lax
from jax.experimental import pallas as pl
from jax.experimental.pallas import tpu as pltpu
from jax.experimental.pallas import tpu_sc as plsc

_B, _L, _D = 1024, 200, 32
_SAMPLE_NUM = 10
_LM1 = _L - 1
_NNEG = _LM1 * _SAMPLE_NUM

_NC = 2
_NS = 16
_NW = _NC * _NS


def _sample_idx_table(L, sample_num, seed=0):
    rng = np.random.RandomState(seed)
    all_idx = [
        rng.choice([j for j in range(L) if j != idx_], size=sample_num, replace=False)
        for idx_ in range(L)
    ]
    return np.stack(all_idx[1:], axis=0).astype(np.int32)


def _routing_tables():
    flat = _sample_idx_table(_L, _SAMPLE_NUM).reshape(-1)
    counts = np.bincount(flat, minlength=_L).astype(np.int32)
    assert int(counts.max()) <= 31
    dtbl = np.zeros((_L, 32), np.int32)
    dtbl[:, 0] = counts
    fill = np.ones((_L,), np.int32)
    for k, j in enumerate(flat):
        dtbl[j, fill[j]] = k
        fill[j] += 1

    weight = counts + (np.arange(_L) >= 1)
    order = np.argsort(-weight, kind="stable")
    loads = np.zeros(_NW, np.int64)
    assign = [[] for _ in range(_NW)]
    for j in order:
        w = int(np.argmin(loads))
        loads[w] += int(weight[j])
        assign[w].append(int(j))
    max_slabs = max(len(a) for a in assign)
    assert max_slabs <= 15
    wtbl = np.zeros((_NW, 16), np.int32)
    for w, a in enumerate(assign):
        wtbl[w, 0] = len(a)
        wtbl[w, 1 : 1 + len(a)] = a
    return dtbl, wtbl


_DTBL, _WTBL = _routing_tables()

_mesh = plsc.VectorSubcoreMesh(core_axis_name="c", subcore_axis_name="s")


@functools.partial(
    pl.kernel,
    mesh=_mesh,
    out_type=(
        jax.ShapeDtypeStruct((_LM1, _D, _B), jnp.float32),
        jax.ShapeDtypeStruct((_NNEG, _D, _B), jnp.float32),
    ),
    scratch_types=[
        pltpu.VMEM((_L, 32), jnp.int32),
        pltpu.VMEM((_NW, 16), jnp.int32),
        pltpu.VMEM((2, _D, _B), jnp.float32),
        pltpu.SemaphoreType.DMA,
        pltpu.SemaphoreType.DMA,
        pltpu.SemaphoreType.DMA,
        pltpu.SemaphoreType.DMA,
    ],
    compiler_params=pltpu.CompilerParams(needs_layout_passes=False),
)
def _sc_route(
    x_hbm, dtbl_hbm, wtbl_hbm, pos_hbm, neg_hbm,
    dtbl_v, wtbl_v, slab_v, lsem0, lsem1, wsem0, wsem1,
):
    w = lax.axis_index("s") * _NC + lax.axis_index("c")
    pltpu.sync_copy(dtbl_hbm, dtbl_v)
    pltpu.sync_copy(wtbl_hbm, wtbl_v)
    lanes = lax.broadcasted_iota(jnp.int32, (16,), 0)
    wv = wtbl_v[w, pl.ds(0, 16)]

    def lane_of(vec, lane):
        return jnp.sum(jnp.where(lanes == lane, vec, 0))

    nslab = lane_of(wv, 0)
    j0 = lane_of(wv, 1)
    pltpu.async_copy(x_hbm.at[j0], slab_v.at[0], lsem0)

    def drain(sem, n):
        def one(_, c):
            pltpu.make_async_copy(slab_v.at[0], neg_hbm.at[0], sem).wait()
            return c

        lax.fori_loop(0, n, one, 0)

    def slab_body(si, prev_writes):
        p = si % 2
        j = lane_of(wv, si + 1)
        rv1 = dtbl_v[j, pl.ds(0, 16)]
        rv2 = dtbl_v[j, pl.ds(16, 16)]
        cnt = lane_of(rv1, 0)

        @pl.when(p == 0)
        def _():
            pltpu.make_async_copy(x_hbm.at[j], slab_v.at[0], lsem0).wait()

        @pl.when(p == 1)
        def _():
            pltpu.make_async_copy(x_hbm.at[j], slab_v.at[1], lsem1).wait()

        def dest_body(c, carry2):
            cc = c + 1
            rv = jnp.where(cc < 16, rv1, rv2)
            lane = jnp.where(cc < 16, cc, cc - 16)
            d = jnp.sum(jnp.where(lanes == lane, rv, 0))

            @pl.when(p == 0)
            def _():
                pltpu.async_copy(slab_v.at[0], neg_hbm.at[d], wsem0)

            @pl.when(p == 1)
            def _():
                pltpu.async_copy(slab_v.at[1], neg_hbm.at[d], wsem1)

            return carry2

        lax.fori_loop(0, cnt, dest_body, 0)

        @pl.when((j >= 1) & (p == 0))
        def _():
            pltpu.async_copy(slab_v.at[0], pos_hbm.at[j - 1], wsem0)

        @pl.when((j >= 1) & (p == 1))
        def _():
            pltpu.async_copy(slab_v.at[1], pos_hbm.at[j - 1], wsem1)

        writes = cnt + jnp.where(j >= 1, 1, 0)

        @pl.when(p == 0)
        def _():
            drain(wsem1, prev_writes)

        @pl.when(p == 1)
        def _():
            drain(wsem0, prev_writes)

        @pl.when(si + 1 < nslab)
        def _():
            jn = lane_of(wv, si + 2)

            @pl.when(p == 0)
            def _():
                pltpu.async_copy(x_hbm.at[jn], slab_v.at[1], lsem1)

            @pl.when(p == 1)
            def _():
                pltpu.async_copy(x_hbm.at[jn], slab_v.at[0], lsem0)

        return writes

    last_writes = lax.fori_loop(0, nslab, slab_body, 0)

    @pl.when((nslab % 2) == 1)
    def _():
        drain(wsem0, last_writes)

    @pl.when((nslab % 2) == 0)
    def _():
        drain(wsem1, last_writes)


def kernel(inputs):
    b, l, d = inputs.shape
    x_t = jnp.transpose(inputs, (1, 2, 0))
    pos_t, neg_t = _sc_route(x_t, jnp.asarray(_DTBL), jnp.asarray(_WTBL))
    pos = jnp.transpose(pos_t, (2, 0, 1))
    neg = jnp.transpose(
        neg_t.reshape(_LM1, _SAMPLE_NUM, d, b), (3, 0, 1, 2)
    )
    return pos, neg
